# baseline (speedup 1.0000x reference)
import functools

import jax
import jax.numpy as jnp
from jax import lax
from jax.experimental import pallas as pl
from jax.experimental.pallas import tpu as pltpu
from jax.experimental.pallas import tpu_sc as plsc

_NC, _NS, _L = 2, 16, 16
_NW = _NC * _NS

_BS = 2048


def _tc_body(x_ref, e_ref, o_ref):
    o_ref[...] = x_ref[...] + e_ref[...][None]


def _tc_add(x, emb_weight):
    B, S, D = x.shape
    grid = (S // _BS, B)
    return pl.pallas_call(
        _tc_body,
        grid=grid,
        in_specs=[
            pl.BlockSpec((1, _BS, D), lambda i, b: (b, i, 0)),
            pl.BlockSpec((_BS, D), lambda i, b: (i, 0)),
        ],
        out_specs=pl.BlockSpec((1, _BS, D), lambda i, b: (b, i, 0)),
        out_shape=jax.ShapeDtypeStruct(x.shape, x.dtype),
        compiler_params=pltpu.CompilerParams(
            dimension_semantics=("arbitrary", "arbitrary")
        ),
    )(x, emb_weight)


def _sc_add(x, emb_weight):
    B, S, D = x.shape
    rows_w = S // _NW
    R = 16
    n_chunks = rows_w // R
    mesh = plsc.VectorSubcoreMesh(
        core_axis_name="c", subcore_axis_name="s",
        num_cores=_NC, num_subcores=_NS,
    )

    @functools.partial(
        pl.kernel,
        out_type=jax.ShapeDtypeStruct((B, S, D), jnp.float32),
        mesh=mesh,
        scratch_types=[
            pltpu.VMEM((R, D), jnp.float32),
            pltpu.VMEM((R, D), jnp.float32),
        ],
    )
    def k(x_hbm, emb_hbm, out_hbm, emb_v, x_v):
        wid = lax.axis_index("s") * _NC + lax.axis_index("c")
        s_base = wid * rows_w

        def chunk(ci, carry):
            s0 = s_base + ci * R
            pltpu.sync_copy(emb_hbm.at[pl.ds(s0, R), :], emb_v)
            for b in range(B):
                pltpu.sync_copy(x_hbm.at[b, pl.ds(s0, R), :], x_v)

                def col(j, c2):
                    for r in range(R):
                        sl = pl.ds(j * _L, _L)
                        x_v[r, sl] = x_v[r, sl] + emb_v[r, sl]
                    return c2

                lax.fori_loop(0, D // _L, col, 0)
                pltpu.sync_copy(x_v, out_hbm.at[b, pl.ds(s0, R), :])
            return carry

        lax.fori_loop(0, n_chunks, chunk, 0)

    return k(x, emb_weight)


def kernel(x, emb_weight):
    return _sc_add(x, emb_weight)


# speedup vs baseline: 1.5021x; 1.5021x over previous
import functools

import jax
import jax.numpy as jnp
from jax import lax
from jax.experimental import pallas as pl
from jax.experimental.pallas import tpu as pltpu
from jax.experimental.pallas import tpu_sc as plsc

_NC, _NS, _L = 2, 16, 16
_NW = _NC * _NS

_BS = 2048
_S1 = 6144


def _tc_body(x_ref, e_ref, o_ref):
    o_ref[...] = x_ref[...] + e_ref[...][None]


def _tc_add(x, emb_weight, s_hi):
    B, S, D = x.shape
    bs = min(_BS, s_hi)
    grid = (s_hi // bs, B)
    return pl.pallas_call(
        _tc_body,
        grid=grid,
        in_specs=[
            pl.BlockSpec((1, bs, D), lambda i, b: (b, i, 0)),
            pl.BlockSpec((bs, D), lambda i, b: (i, 0)),
        ],
        out_specs=pl.BlockSpec((1, bs, D), lambda i, b: (b, i, 0)),
        out_shape=jax.ShapeDtypeStruct((B, s_hi, D), x.dtype),
        compiler_params=pltpu.CompilerParams(
            dimension_semantics=("arbitrary", "arbitrary")
        ),
    )(x, emb_weight)


def _sc_add(x, emb_weight, s_lo):
    B, S, D = x.shape
    n_rows = S - s_lo
    rows_w = n_rows // _NW
    R = min(16, rows_w)
    n_chunks = rows_w // R
    mesh = plsc.VectorSubcoreMesh(
        core_axis_name="c", subcore_axis_name="s",
        num_cores=_NC, num_subcores=_NS,
    )

    @functools.partial(
        pl.kernel,
        out_type=jax.ShapeDtypeStruct((B, n_rows, D), jnp.float32),
        mesh=mesh,
        scratch_types=[
            pltpu.VMEM((R, D), jnp.float32),
            pltpu.VMEM((R, D), jnp.float32),
        ],
    )
    def k(x_hbm, emb_hbm, out_hbm, emb_v, x_v):
        wid = lax.axis_index("s") * _NC + lax.axis_index("c")
        r_base = wid * rows_w

        def chunk(ci, carry):
            r0 = r_base + ci * R
            s0 = s_lo + r0
            pltpu.sync_copy(emb_hbm.at[pl.ds(s0, R), :], emb_v)
            for b in range(B):
                pltpu.sync_copy(x_hbm.at[b, pl.ds(s0, R), :], x_v)

                def col(j, c2):
                    for r in range(R):
                        sl = pl.ds(j * _L, _L)
                        x_v[r, sl] = x_v[r, sl] + emb_v[r, sl]
                    return c2

                lax.fori_loop(0, D // _L, col, 0)
                pltpu.sync_copy(x_v, out_hbm.at[b, pl.ds(r0, R), :])
            return carry

        lax.fori_loop(0, n_chunks, chunk, 0)

    return k(x, emb_weight)


def kernel(x, emb_weight):
    tc_out = _tc_add(x, emb_weight, _S1)
    sc_out = _sc_add(x, emb_weight, _S1)
    return jnp.concatenate([tc_out, sc_out], axis=1)


# speedup vs baseline: 1.8308x; 1.2189x over previous
import functools

import jax
import jax.numpy as jnp
from jax import lax
from jax.experimental import pallas as pl
from jax.experimental.pallas import tpu as pltpu
from jax.experimental.pallas import tpu_sc as plsc

_NC, _NS, _L = 2, 16, 16
_NW = _NC * _NS

_BS = 2048


def _tc_body(x_ref, e_ref, o_ref):
    o_ref[...] = x_ref[...] + e_ref[...][None]


def _tc_add(x, emb_weight):
    B, S, D = x.shape
    grid = (S // _BS, B)
    return pl.pallas_call(
        _tc_body,
        grid=grid,
        in_specs=[
            pl.BlockSpec((1, _BS, D), lambda i, b: (b, i, 0)),
            pl.BlockSpec((_BS, D), lambda i, b: (i, 0)),
        ],
        out_specs=pl.BlockSpec((1, _BS, D), lambda i, b: (b, i, 0)),
        out_shape=jax.ShapeDtypeStruct(x.shape, x.dtype),
        compiler_params=pltpu.CompilerParams(
            dimension_semantics=("arbitrary", "arbitrary")
        ),
    )(x, emb_weight)


def _sc_add(x, emb_weight):
    B, S, D = x.shape
    rows_w = S // _NW
    R = 8
    n_chunks = rows_w // R
    n_cols = D // _L
    mesh = plsc.VectorSubcoreMesh(
        core_axis_name="c", subcore_axis_name="s",
        num_cores=_NC, num_subcores=_NS,
    )

    @functools.partial(
        pl.kernel,
        out_type=jax.ShapeDtypeStruct((B, S, D), jnp.float32),
        mesh=mesh,
        scratch_types=[
            pltpu.VMEM((B, R, D), jnp.float32),
            pltpu.VMEM((B, R, D), jnp.float32),
            pltpu.VMEM((2, R, D), jnp.float32),
            pltpu.SemaphoreType.DMA((B,)),
            pltpu.SemaphoreType.DMA((B,)),
            pltpu.SemaphoreType.DMA((2,)),
        ],
    )
    def k(x_hbm, emb_hbm, out_hbm, xin, xout, emb2, in_sem, out_sem, emb_sem):
        wid = lax.axis_index("s") * _NC + lax.axis_index("c")
        r_base = wid * rows_w

        def fire_in(ci, b):
            pltpu.async_copy(
                x_hbm.at[b, pl.ds(r_base + ci * R, R), :], xin.at[b],
                in_sem.at[b])

        def wait_in(b):
            pltpu.make_async_copy(
                x_hbm.at[b, pl.ds(r_base, R), :], xin.at[b],
                in_sem.at[b]).wait()

        def fire_out(ci, b):
            pltpu.async_copy(
                xout.at[b], out_hbm.at[b, pl.ds(r_base + ci * R, R), :],
                out_sem.at[b])

        def wait_out(b):
            pltpu.make_async_copy(
                xout.at[b], out_hbm.at[b, pl.ds(r_base, R), :],
                out_sem.at[b]).wait()

        def fire_emb(ci, slot):
            pltpu.async_copy(
                emb_hbm.at[pl.ds(r_base + ci * R, R), :], emb2.at[slot],
                emb_sem.at[slot])

        def wait_emb(slot):
            pltpu.make_async_copy(
                emb_hbm.at[pl.ds(r_base, R), :], emb2.at[slot],
                emb_sem.at[slot]).wait()

        def do_chunk(ci, slot, drain_out):
            wait_emb(slot)
            for b in range(B):
                wait_in(b)
                if drain_out:
                    wait_out(b)

                def col(j, c2, b=b, slot=slot):
                    for r in range(R):
                        sl = pl.ds(j * _L, _L)
                        xout[b, r, sl] = xin[b, r, sl] + emb2[slot, r, sl]
                    return c2

                lax.fori_loop(0, n_cols, col, 0)

                @pl.when(ci + 1 < n_chunks)
                def _(ci=ci, b=b):
                    fire_in(ci + 1, b)

                fire_out(ci, b)

            @pl.when(ci + 2 < n_chunks)
            def _(ci=ci, slot=slot):
                fire_emb(ci + 2, slot)

        fire_emb(0, 0)
        fire_emb(1, 1)
        for b in range(B):
            fire_in(0, b)

        do_chunk(0, 0, drain_out=False)
        do_chunk(1, 1, drain_out=True)

        def two_chunks(i, carry):
            do_chunk(2 * i, 0, drain_out=True)
            do_chunk(2 * i + 1, 1, drain_out=True)
            return carry

        lax.fori_loop(1, n_chunks // 2, two_chunks, 0)

        for b in range(B):
            wait_out(b)

    return k(x, emb_weight)


def kernel(x, emb_weight):
    return _sc_add(x, emb_weight)


# speedup vs baseline: 3.2781x; 1.7905x over previous
import jax
import jax.numpy as jnp
from jax.experimental import pallas as pl
from jax.experimental.pallas import tpu as pltpu

_BS = 2048


def _add_body(x_ref, e_ref, o_ref):
    o_ref[...] = x_ref[...] + e_ref[...][None]


def kernel(x, emb_weight):
    B, S, D = x.shape
    grid = (S // _BS, B)
    return pl.pallas_call(
        _add_body,
        grid=grid,
        in_specs=[
            pl.BlockSpec((1, _BS, D), lambda i, b: (b, i, 0)),
            pl.BlockSpec((_BS, D), lambda i, b: (i, 0)),
        ],
        out_specs=pl.BlockSpec((1, _BS, D), lambda i, b: (b, i, 0)),
        out_shape=jax.ShapeDtypeStruct(x.shape, x.dtype),
        compiler_params=pltpu.CompilerParams(
            dimension_semantics=("arbitrary", "arbitrary")
        ),
    )(x, emb_weight)
